# baseline (speedup 1.0000x reference)
import functools

import jax
import jax.numpy as jnp
from jax import lax
from jax.experimental import pallas as pl
from jax.experimental.pallas import tpu as pltpu
from jax.experimental.pallas import tpu_sc as plsc

N = 10000
NP = 10240
E = 320000
NCORES = 2
NSUB = 16
NTILES = NCORES * NSUB
CHUNKS = 80
CW = 128
EP = NTILES * CHUNKS * CW
F_IN = 128
HID = 16
C_OUT = 16
SLICE_PER_SUB = NP // NSUB
NBUF = 4

_sc_mesh = plsc.VectorSubcoreMesh(core_axis_name="c", subcore_axis_name="s")
_sc_params = pltpu.CompilerParams(
    needs_layout_passes=False, use_tc_tiling_on_sc=False)


def _rsqrt16(d):
    i = jnp.int32(0x5F3759DF) - (plsc.bitcast(d, jnp.int32) >> 1)
    y = plsc.bitcast(i, jnp.float32)
    for _ in range(3):
        y = y * (1.5 - 0.5 * d * y * y)
    return y


def _zero_rows(ref, n):
    z16 = jnp.zeros((16,), jnp.float32)

    def body(i, _):
        ref[i] = z16
        return 0

    lax.fori_loop(0, n, body, 0)


def _zero_flat(ref, n16):
    z16 = jnp.zeros((16,), jnp.float32)

    def body(i, _):
        ref[pl.ds(i * 16, 16)] = z16
        return 0

    lax.fori_loop(0, n16, body, 0)


def _agg_pipeline(s, colb, wb, row_of, v_hbm, sh_agg, rows_v, sems, zb):
    def za_body(j, _):
        pltpu.sync_copy(zb, sh_agg.at[pl.ds(s * SLICE_PER_SUB + j * CW, CW)])
        return 0

    lax.fori_loop(0, SLICE_PER_SUB // CW, za_body, 0)
    plsc.subcore_barrier()

    for b in range(NBUF):
        pltpu.async_copy(v_hbm.at[colb.at[b]], rows_v.at[b], sems.at[b])

    def body(i, _):
        for b in range(NBUF):
            jj = i * NBUF + b
            pltpu.make_async_copy(
                v_hbm.at[colb.at[jj]], rows_v.at[b], sems.at[b]).wait()
            for k in range(CW // 16):
                w16 = wb[jj, pl.ds(k * 16, 16)]
                base = k * 16
                for g in range(16):
                    wg = jnp.broadcast_to(w16[g], (16,))
                    rows_v[b, base + g] = rows_v[b, base + g] * wg
            pltpu.sync_copy(rows_v.at[b], sh_agg.at[row_of(jj)], add=True)

            @pl.when(jj + NBUF < CHUNKS)
            def _():
                pltpu.async_copy(
                    v_hbm.at[colb.at[jj + NBUF]], rows_v.at[b], sems.at[b])
        return 0

    lax.fori_loop(0, CHUNKS // NBUF, body, 0)
    plsc.subcore_barrier()


def _agg_writeout(c, s, sh_agg, agg_hbm):
    def body(j, _):
        r0 = s * SLICE_PER_SUB + j * CW
        pltpu.sync_copy(sh_agg.at[pl.ds(r0, CW)], agg_hbm.at[c, pl.ds(r0, CW)])
        return 0

    lax.fori_loop(0, SLICE_PER_SUB // CW, body, 0)


@functools.partial(
    pl.kernel,
    out_type=(
        jax.ShapeDtypeStruct((NTILES, CHUNKS, CW), jnp.float32),
        jax.ShapeDtypeStruct((NCORES, NP, HID), jnp.float32),
    ),
    mesh=_sc_mesh,
    compiler_params=_sc_params,
    scratch_types=(
        pltpu.VMEM((2, CHUNKS, CW), jnp.int32),
        pltpu.VMEM((2, CHUNKS, CW), jnp.float32),
        pltpu.VMEM((CHUNKS, CW), jnp.int32),
        pltpu.VMEM((CHUNKS, CW), jnp.float32),
        pltpu.VMEM((NP,), jnp.float32),
        pltpu.VMEM((NP,), jnp.float32),
        pltpu.VMEM((SLICE_PER_SUB,), jnp.float32),
        pltpu.VMEM((SLICE_PER_SUB,), jnp.float32),
        pltpu.VMEM((SLICE_PER_SUB,), jnp.float32),
        pltpu.VMEM((CW, 16), jnp.float32),
        pltpu.VMEM((NBUF, CW, HID), jnp.float32),
        pltpu.VMEM_SHARED((NSUB, NP), jnp.float32),
        pltpu.VMEM_SHARED((NP,), jnp.float32),
        pltpu.VMEM_SHARED((NP, HID), jnp.float32),
        pltpu.SemaphoreType.DMA((NBUF,)),
    ),
)
def _sc_prep_agg1(row_hbm, col_hbm, ew_hbm, v_hbm, wn_hbm, agg_hbm,
                  row2, ew2, colb, wb, degl, disb, acc, tmpd, disc, zb,
                  rows_v, sh_slots, sh_dis, sh_agg, sems):
    c = lax.axis_index("c")
    s = lax.axis_index("s")
    wid = c * NSUB + s

    pltpu.sync_copy(row_hbm.at[s], row2.at[0])
    pltpu.sync_copy(row_hbm.at[s + NSUB], row2.at[1])
    pltpu.sync_copy(ew_hbm.at[s], ew2.at[0])
    pltpu.sync_copy(ew_hbm.at[s + NSUB], ew2.at[1])
    pltpu.sync_copy(col_hbm.at[wid], colb)

    _zero_rows(zb, CW)
    _zero_flat(degl, NP // 16)

    def deg_body(j, _):
        for t in range(2):
            for k in range(8):
                sl = pl.ds(k * 16, 16)
                plsc.addupdate_scatter(degl, [row2[t, j, sl]], ew2[t, j, sl])
        return 0

    lax.fori_loop(0, CHUNKS, deg_body, 0)

    pltpu.sync_copy(degl, sh_slots.at[s])
    plsc.subcore_barrier()

    base = s * SLICE_PER_SUB
    _zero_flat(acc, SLICE_PER_SUB // 16)

    def red_body(t, _):
        pltpu.sync_copy(sh_slots.at[t, pl.ds(base, SLICE_PER_SUB)], tmpd)

        def add_body(r, _):
            sl = pl.ds(r * 16, 16)
            acc[sl] = acc[sl] + tmpd[sl]
            return 0

        lax.fori_loop(0, SLICE_PER_SUB // 16, add_body, 0)
        return 0

    lax.fori_loop(0, NSUB, red_body, 0)

    def dis_body(r, _):
        sl = pl.ds(r * 16, 16)
        d = acc[sl]
        y = _rsqrt16(jnp.maximum(d, 1e-30))
        disc[sl] = jnp.where(d > 0, y, 0.0)
        return 0

    lax.fori_loop(0, SLICE_PER_SUB // 16, dis_body, 0)
    pltpu.sync_copy(disc, sh_dis.at[pl.ds(base, SLICE_PER_SUB)])
    plsc.subcore_barrier()
    pltpu.sync_copy(sh_dis, disb)

    def wn_body(j, _):
        for k in range(8):
            sl = pl.ds(k * 16, 16)
            dr = plsc.load_gather(disb, [row2[c, j, sl]])
            dc = plsc.load_gather(disb, [colb[j, sl]])
            wb[j, sl] = dr * ew2[c, j, sl] * dc
        return 0

    lax.fori_loop(0, CHUNKS, wn_body, 0)
    pltpu.sync_copy(wb, wn_hbm.at[wid])

    _agg_pipeline(s, colb, wb, lambda jj: row2.at[c, jj],
                  v_hbm, sh_agg, rows_v, sems, zb)
    _agg_writeout(c, s, sh_agg, agg_hbm)


@functools.partial(
    pl.kernel,
    out_type=jax.ShapeDtypeStruct((NCORES, NP, HID), jnp.float32),
    mesh=_sc_mesh,
    compiler_params=_sc_params,
    scratch_types=(
        pltpu.VMEM((CHUNKS, CW), jnp.int32),
        pltpu.VMEM((CHUNKS, CW), jnp.int32),
        pltpu.VMEM((CHUNKS, CW), jnp.float32),
        pltpu.VMEM((CW, 16), jnp.float32),
        pltpu.VMEM((NBUF, CW, HID), jnp.float32),
        pltpu.VMEM_SHARED((NP, HID), jnp.float32),
        pltpu.SemaphoreType.DMA((NBUF,)),
    ),
)
def _sc_agg(row_hbm, col_hbm, wn_hbm, v_hbm, agg_hbm,
            rowb, colb, wb, zb, rows_v, sh_agg, sems):
    c = lax.axis_index("c")
    s = lax.axis_index("s")
    wid = c * NSUB + s

    pltpu.sync_copy(row_hbm.at[wid], rowb)
    pltpu.sync_copy(col_hbm.at[wid], colb)
    pltpu.sync_copy(wn_hbm.at[wid], wb)
    _zero_rows(zb, CW)

    _agg_pipeline(s, colb, wb, lambda jj: rowb.at[jj],
                  v_hbm, sh_agg, rows_v, sems, zb)
    _agg_writeout(c, s, sh_agg, agg_hbm)


def _mm1_body(x_ref, w0_ref, w1_ref, y0_ref, y1_ref):
    x = x_ref[...]
    y0_ref[...] = jnp.dot(x, w0_ref[...], preferred_element_type=jnp.float32)
    y1_ref[...] = jnp.dot(x, w1_ref[...], preferred_element_type=jnp.float32)


def _mid_body(y0_ref, agg_ref, b_ref, w0_ref, w1_ref, z0_ref, z1_ref):
    p = agg_ref[0] + agg_ref[1]
    h = jnp.maximum(y0_ref[...] - p + b_ref[0:1, :], 0.0)
    z0_ref[...] = jnp.dot(h, w0_ref[...], preferred_element_type=jnp.float32)
    z1_ref[...] = jnp.dot(h, w1_ref[...], preferred_element_type=jnp.float32)


def _fin_body(z0_ref, agg_ref, b_ref, out_ref):
    o = z0_ref[...] - (agg_ref[0] + agg_ref[1]) + b_ref[0:1, :]
    m = jnp.max(o, axis=1, keepdims=True)
    ex = jnp.exp(o - m)
    out_ref[...] = o - m - jnp.log(jnp.sum(ex, axis=1, keepdims=True))


_RB = 1000


def kernel(x, edge_index, edge_weight, W1, b1, W2, b2):
    row = edge_index[0]
    col = edge_index[1]
    pad = EP - E
    zpad_i = jnp.zeros((pad,), row.dtype)
    rowp = jnp.concatenate([row, zpad_i]).reshape(NTILES, CHUNKS, CW)
    colp = jnp.concatenate([col, zpad_i]).reshape(NTILES, CHUNKS, CW)
    ewp = jnp.concatenate([edge_weight, jnp.zeros((pad,), edge_weight.dtype)])
    ewp = ewp.reshape(NTILES, CHUNKS, CW)
    b1b = jnp.broadcast_to(b1.reshape(1, HID), (8, HID))
    b2b = jnp.broadcast_to(b2.reshape(1, C_OUT), (8, C_OUT))

    grid = N // _RB
    y0, y1 = pl.pallas_call(
        _mm1_body,
        grid=(grid,),
        in_specs=[
            pl.BlockSpec((_RB, F_IN), lambda i: (i, 0)),
            pl.BlockSpec((F_IN, HID), lambda i: (0, 0)),
            pl.BlockSpec((F_IN, HID), lambda i: (0, 0)),
        ],
        out_specs=[
            pl.BlockSpec((_RB, HID), lambda i: (i, 0)),
            pl.BlockSpec((_RB, HID), lambda i: (i, 0)),
        ],
        out_shape=[
            jax.ShapeDtypeStruct((N, HID), jnp.float32),
            jax.ShapeDtypeStruct((N, HID), jnp.float32),
        ],
    )(x, W1[0], W1[1])

    wn, agg1 = _sc_prep_agg1(rowp, colp, ewp, y1)

    z0, z1 = pl.pallas_call(
        _mid_body,
        grid=(grid,),
        in_specs=[
            pl.BlockSpec((_RB, HID), lambda i: (i, 0)),
            pl.BlockSpec((NCORES, _RB, HID), lambda i: (0, i, 0)),
            pl.BlockSpec((8, HID), lambda i: (0, 0)),
            pl.BlockSpec((HID, C_OUT), lambda i: (0, 0)),
            pl.BlockSpec((HID, C_OUT), lambda i: (0, 0)),
        ],
        out_specs=[
            pl.BlockSpec((_RB, C_OUT), lambda i: (i, 0)),
            pl.BlockSpec((_RB, C_OUT), lambda i: (i, 0)),
        ],
        out_shape=[
            jax.ShapeDtypeStruct((N, C_OUT), jnp.float32),
            jax.ShapeDtypeStruct((N, C_OUT), jnp.float32),
        ],
    )(y0, agg1, b1b, W2[0], W2[1])

    agg2 = _sc_agg(rowp, colp, wn, z1)

    out = pl.pallas_call(
        _fin_body,
        grid=(grid,),
        in_specs=[
            pl.BlockSpec((_RB, C_OUT), lambda i: (i, 0)),
            pl.BlockSpec((NCORES, _RB, C_OUT), lambda i: (0, i, 0)),
            pl.BlockSpec((8, C_OUT), lambda i: (0, 0)),
        ],
        out_specs=pl.BlockSpec((_RB, C_OUT), lambda i: (i, 0)),
        out_shape=jax.ShapeDtypeStruct((N, C_OUT), jnp.float32),
    )(z0, agg2, b2b)
    return out


# speedup vs baseline: 1.0568x; 1.0568x over previous
import functools

import jax
import jax.numpy as jnp
from jax import lax
from jax.experimental import pallas as pl
from jax.experimental.pallas import tpu as pltpu
from jax.experimental.pallas import tpu_sc as plsc

N = 10000
NP = 10240
E = 320000
NCORES = 2
NSUB = 16
NTILES = NCORES * NSUB
CHUNKS = 80
CW = 128
EP = NTILES * CHUNKS * CW
F_IN = 128
HID = 16
C_OUT = 16
SLICE_PER_SUB = NP // NSUB
NBUF = 4

_sc_mesh = plsc.VectorSubcoreMesh(core_axis_name="c", subcore_axis_name="s")
_sc_params = pltpu.CompilerParams(
    needs_layout_passes=False, use_tc_tiling_on_sc=False)


def _rsqrt16(d):
    i = jnp.int32(0x5F3759DF) - (plsc.bitcast(d, jnp.int32) >> 1)
    y = plsc.bitcast(i, jnp.float32)
    for _ in range(3):
        y = y * (1.5 - 0.5 * d * y * y)
    return y


def _zero_rows(ref, n):
    z16 = jnp.zeros((16,), jnp.float32)

    def body(i, _):
        ref[i] = z16
        return 0

    lax.fori_loop(0, n, body, 0)


def _zero_flat(ref, n16):
    z16 = jnp.zeros((16,), jnp.float32)

    def body(i, _):
        ref[pl.ds(i * 16, 16)] = z16
        return 0

    lax.fori_loop(0, n16, body, 0)


def _agg_pipeline(s, colb, wb, row_of, v_hbm, sh_agg, rows_v, sems, zb):
    def za_body(j, _):
        pltpu.sync_copy(zb, sh_agg.at[pl.ds(s * SLICE_PER_SUB + j * CW, CW)])
        return 0

    lax.fori_loop(0, SLICE_PER_SUB // CW, za_body, 0)
    plsc.subcore_barrier()

    for b in range(NBUF):
        pltpu.async_copy(v_hbm.at[colb.at[b]], rows_v.at[b], sems.at[b])

    def body(i, _):
        for b in range(NBUF):
            jj = i * NBUF + b
            pltpu.make_async_copy(
                v_hbm.at[colb.at[jj]], rows_v.at[b], sems.at[b]).wait()
            for k in range(CW // 16):
                w16 = wb[jj, pl.ds(k * 16, 16)]
                base = k * 16
                for g in range(16):
                    wg = jnp.broadcast_to(w16[g], (16,))
                    rows_v[b, base + g] = rows_v[b, base + g] * wg
            pltpu.sync_copy(rows_v.at[b], sh_agg.at[row_of(jj)], add=True)

            @pl.when(jj + NBUF < CHUNKS)
            def _():
                pltpu.async_copy(
                    v_hbm.at[colb.at[jj + NBUF]], rows_v.at[b], sems.at[b])
        return 0

    lax.fori_loop(0, CHUNKS // NBUF, body, 0)
    plsc.subcore_barrier()


def _agg_writeout(c, s, sh_agg, agg_hbm):
    def body(j, _):
        r0 = s * SLICE_PER_SUB + j * CW
        pltpu.sync_copy(sh_agg.at[pl.ds(r0, CW)], agg_hbm.at[c, pl.ds(r0, CW)])
        return 0

    lax.fori_loop(0, SLICE_PER_SUB // CW, body, 0)


@functools.partial(
    pl.kernel,
    out_type=jax.ShapeDtypeStruct((NTILES, CHUNKS, CW), jnp.float32),
    mesh=_sc_mesh,
    compiler_params=_sc_params,
    scratch_types=(
        pltpu.VMEM((2, CHUNKS, CW), jnp.int32),
        pltpu.VMEM((2, CHUNKS, CW), jnp.float32),
        pltpu.VMEM((CHUNKS, CW), jnp.int32),
        pltpu.VMEM((CHUNKS, CW), jnp.float32),
        pltpu.VMEM((NP,), jnp.float32),
        pltpu.VMEM((NP,), jnp.float32),
        pltpu.VMEM((SLICE_PER_SUB,), jnp.float32),
        pltpu.VMEM((SLICE_PER_SUB,), jnp.float32),
        pltpu.VMEM((SLICE_PER_SUB,), jnp.float32),
        pltpu.VMEM_SHARED((NSUB, NP), jnp.float32),
        pltpu.VMEM_SHARED((NP,), jnp.float32),
    ),
)
def _sc_prep(row_hbm, col_hbm, ew_hbm, wn_hbm,
             row2, ew2, colb, wb, degl, disb, acc, tmpd, disc,
             sh_slots, sh_dis):
    c = lax.axis_index("c")
    s = lax.axis_index("s")
    wid = c * NSUB + s

    pltpu.sync_copy(row_hbm.at[s], row2.at[0])
    pltpu.sync_copy(row_hbm.at[s + NSUB], row2.at[1])
    pltpu.sync_copy(ew_hbm.at[s], ew2.at[0])
    pltpu.sync_copy(ew_hbm.at[s + NSUB], ew2.at[1])
    pltpu.sync_copy(col_hbm.at[wid], colb)

    _zero_flat(degl, NP // 16)

    def deg_body(j, _):
        for t in range(2):
            for k in range(8):
                sl = pl.ds(k * 16, 16)
                plsc.addupdate_scatter(degl, [row2[t, j, sl]], ew2[t, j, sl])
        return 0

    lax.fori_loop(0, CHUNKS, deg_body, 0)

    pltpu.sync_copy(degl, sh_slots.at[s])
    plsc.subcore_barrier()

    base = s * SLICE_PER_SUB
    _zero_flat(acc, SLICE_PER_SUB // 16)

    def red_body(t, _):
        pltpu.sync_copy(sh_slots.at[t, pl.ds(base, SLICE_PER_SUB)], tmpd)

        def add_body(r, _):
            sl = pl.ds(r * 16, 16)
            acc[sl] = acc[sl] + tmpd[sl]
            return 0

        lax.fori_loop(0, SLICE_PER_SUB // 16, add_body, 0)
        return 0

    lax.fori_loop(0, NSUB, red_body, 0)

    def dis_body(r, _):
        sl = pl.ds(r * 16, 16)
        d = acc[sl]
        y = _rsqrt16(jnp.maximum(d, 1e-30))
        disc[sl] = jnp.where(d > 0, y, 0.0)
        return 0

    lax.fori_loop(0, SLICE_PER_SUB // 16, dis_body, 0)
    pltpu.sync_copy(disc, sh_dis.at[pl.ds(base, SLICE_PER_SUB)])
    plsc.subcore_barrier()
    pltpu.sync_copy(sh_dis, disb)

    def wn_body(j, _):
        for k in range(8):
            sl = pl.ds(k * 16, 16)
            dr = plsc.load_gather(disb, [row2[c, j, sl]])
            dc = plsc.load_gather(disb, [colb[j, sl]])
            wb[j, sl] = dr * ew2[c, j, sl] * dc
        return 0

    lax.fori_loop(0, CHUNKS, wn_body, 0)
    pltpu.sync_copy(wb, wn_hbm.at[wid])


@functools.partial(
    pl.kernel,
    out_type=jax.ShapeDtypeStruct((NCORES, NP, HID), jnp.float32),
    mesh=_sc_mesh,
    compiler_params=_sc_params,
    scratch_types=(
        pltpu.VMEM((CHUNKS, CW), jnp.int32),
        pltpu.VMEM((CHUNKS, CW), jnp.int32),
        pltpu.VMEM((CHUNKS, CW), jnp.float32),
        pltpu.VMEM((CW, 16), jnp.float32),
        pltpu.VMEM((NBUF, CW, HID), jnp.float32),
        pltpu.VMEM_SHARED((NP, HID), jnp.float32),
        pltpu.SemaphoreType.DMA((NBUF,)),
    ),
)
def _sc_agg(row_hbm, col_hbm, wn_hbm, v_hbm, agg_hbm,
            rowb, colb, wb, zb, rows_v, sh_agg, sems):
    c = lax.axis_index("c")
    s = lax.axis_index("s")
    wid = c * NSUB + s

    pltpu.sync_copy(row_hbm.at[wid], rowb)
    pltpu.sync_copy(col_hbm.at[wid], colb)
    pltpu.sync_copy(wn_hbm.at[wid], wb)
    _zero_rows(zb, CW)

    _agg_pipeline(s, colb, wb, lambda jj: rowb.at[jj],
                  v_hbm, sh_agg, rows_v, sems, zb)
    _agg_writeout(c, s, sh_agg, agg_hbm)


def _mm1_body(x_ref, w0_ref, w1_ref, y0_ref, y1_ref):
    x = x_ref[...]
    y0_ref[...] = jnp.dot(x, w0_ref[...], preferred_element_type=jnp.float32)
    y1_ref[...] = jnp.dot(x, w1_ref[...], preferred_element_type=jnp.float32)


def _mid_body(y0_ref, agg_ref, b_ref, w0_ref, w1_ref, z0_ref, z1_ref):
    p = agg_ref[0] + agg_ref[1]
    h = jnp.maximum(y0_ref[...] - p + b_ref[0:1, :], 0.0)
    z0_ref[...] = jnp.dot(h, w0_ref[...], preferred_element_type=jnp.float32)
    z1_ref[...] = jnp.dot(h, w1_ref[...], preferred_element_type=jnp.float32)


def _fin_body(z0_ref, agg_ref, b_ref, out_ref):
    o = z0_ref[...] - (agg_ref[0] + agg_ref[1]) + b_ref[0:1, :]
    m = jnp.max(o, axis=1, keepdims=True)
    ex = jnp.exp(o - m)
    out_ref[...] = o - m - jnp.log(jnp.sum(ex, axis=1, keepdims=True))


_RB = 1000


def kernel(x, edge_index, edge_weight, W1, b1, W2, b2):
    row = edge_index[0]
    col = edge_index[1]
    pad = EP - E
    zpad_i = jnp.zeros((pad,), row.dtype)
    rowp = jnp.concatenate([row, zpad_i]).reshape(NTILES, CHUNKS, CW)
    colp = jnp.concatenate([col, zpad_i]).reshape(NTILES, CHUNKS, CW)
    ewp = jnp.concatenate([edge_weight, jnp.zeros((pad,), edge_weight.dtype)])
    ewp = ewp.reshape(NTILES, CHUNKS, CW)
    b1b = jnp.broadcast_to(b1.reshape(1, HID), (8, HID))
    b2b = jnp.broadcast_to(b2.reshape(1, C_OUT), (8, C_OUT))

    grid = N // _RB
    y0, y1 = pl.pallas_call(
        _mm1_body,
        grid=(grid,),
        in_specs=[
            pl.BlockSpec((_RB, F_IN), lambda i: (i, 0)),
            pl.BlockSpec((F_IN, HID), lambda i: (0, 0)),
            pl.BlockSpec((F_IN, HID), lambda i: (0, 0)),
        ],
        out_specs=[
            pl.BlockSpec((_RB, HID), lambda i: (i, 0)),
            pl.BlockSpec((_RB, HID), lambda i: (i, 0)),
        ],
        out_shape=[
            jax.ShapeDtypeStruct((N, HID), jnp.float32),
            jax.ShapeDtypeStruct((N, HID), jnp.float32),
        ],
    )(x, W1[0], W1[1])

    wn = _sc_prep(rowp, colp, ewp)
    agg1 = _sc_agg(rowp, colp, wn, y1)

    z0, z1 = pl.pallas_call(
        _mid_body,
        grid=(grid,),
        in_specs=[
            pl.BlockSpec((_RB, HID), lambda i: (i, 0)),
            pl.BlockSpec((NCORES, _RB, HID), lambda i: (0, i, 0)),
            pl.BlockSpec((8, HID), lambda i: (0, 0)),
            pl.BlockSpec((HID, C_OUT), lambda i: (0, 0)),
            pl.BlockSpec((HID, C_OUT), lambda i: (0, 0)),
        ],
        out_specs=[
            pl.BlockSpec((_RB, C_OUT), lambda i: (i, 0)),
            pl.BlockSpec((_RB, C_OUT), lambda i: (i, 0)),
        ],
        out_shape=[
            jax.ShapeDtypeStruct((N, C_OUT), jnp.float32),
            jax.ShapeDtypeStruct((N, C_OUT), jnp.float32),
        ],
    )(y0, agg1, b1b, W2[0], W2[1])

    agg2 = _sc_agg(rowp, colp, wn, z1)

    out = pl.pallas_call(
        _fin_body,
        grid=(grid,),
        in_specs=[
            pl.BlockSpec((_RB, C_OUT), lambda i: (i, 0)),
            pl.BlockSpec((NCORES, _RB, C_OUT), lambda i: (0, i, 0)),
            pl.BlockSpec((8, C_OUT), lambda i: (0, 0)),
        ],
        out_specs=pl.BlockSpec((_RB, C_OUT), lambda i: (i, 0)),
        out_shape=jax.ShapeDtypeStruct((N, C_OUT), jnp.float32),
    )(z0, agg2, b2b)
    return out


# speedup vs baseline: 1.0602x; 1.0032x over previous
import functools

import jax
import jax.numpy as jnp
from jax import lax
from jax.experimental import pallas as pl
from jax.experimental.pallas import tpu as pltpu
from jax.experimental.pallas import tpu_sc as plsc

N = 10000
NP = 10240
E = 320000
NCORES = 2
NSUB = 16
NTILES = NCORES * NSUB
CHUNKS = 80
CW = 128
EP = NTILES * CHUNKS * CW
F_IN = 128
HID = 16
C_OUT = 16
SLICE_PER_SUB = NP // NSUB
NBUF = 8

_sc_mesh = plsc.VectorSubcoreMesh(core_axis_name="c", subcore_axis_name="s")
_sc_params = pltpu.CompilerParams(
    needs_layout_passes=False, use_tc_tiling_on_sc=False)


def _rsqrt16(d):
    i = jnp.int32(0x5F3759DF) - (plsc.bitcast(d, jnp.int32) >> 1)
    y = plsc.bitcast(i, jnp.float32)
    for _ in range(3):
        y = y * (1.5 - 0.5 * d * y * y)
    return y


def _zero_rows(ref, n):
    z16 = jnp.zeros((16,), jnp.float32)

    def body(i, _):
        ref[i] = z16
        return 0

    lax.fori_loop(0, n, body, 0)


def _zero_flat(ref, n16):
    z16 = jnp.zeros((16,), jnp.float32)

    def body(i, _):
        ref[pl.ds(i * 16, 16)] = z16
        return 0

    lax.fori_loop(0, n16, body, 0)


def _agg_pipeline(s, colb, wb, row_of, v_hbm, sh_agg, rows_v, sems, zb):
    def za_body(j, _):
        pltpu.sync_copy(zb, sh_agg.at[pl.ds(s * SLICE_PER_SUB + j * CW, CW)])
        return 0

    lax.fori_loop(0, SLICE_PER_SUB // CW, za_body, 0)
    plsc.subcore_barrier()

    for b in range(NBUF):
        pltpu.async_copy(v_hbm.at[colb.at[b]], rows_v.at[b], sems.at[b])

    def body(i, _):
        for b in range(NBUF):
            jj = i * NBUF + b
            pltpu.make_async_copy(
                v_hbm.at[colb.at[jj]], rows_v.at[b], sems.at[b]).wait()
            for k in range(CW // 16):
                w16 = wb[jj, pl.ds(k * 16, 16)]
                base = k * 16
                for g in range(16):
                    wg = jnp.broadcast_to(w16[g], (16,))
                    rows_v[b, base + g] = rows_v[b, base + g] * wg
            pltpu.sync_copy(rows_v.at[b], sh_agg.at[row_of(jj)], add=True)

            @pl.when(jj + NBUF < CHUNKS)
            def _():
                pltpu.async_copy(
                    v_hbm.at[colb.at[jj + NBUF]], rows_v.at[b], sems.at[b])
        return 0

    lax.fori_loop(0, CHUNKS // NBUF, body, 0)
    plsc.subcore_barrier()


def _agg_writeout(c, s, sh_agg, agg_hbm):
    def body(j, _):
        r0 = s * SLICE_PER_SUB + j * CW
        pltpu.sync_copy(sh_agg.at[pl.ds(r0, CW)], agg_hbm.at[c, pl.ds(r0, CW)])
        return 0

    lax.fori_loop(0, SLICE_PER_SUB // CW, body, 0)


@functools.partial(
    pl.kernel,
    out_type=jax.ShapeDtypeStruct((NTILES, CHUNKS, CW), jnp.float32),
    mesh=_sc_mesh,
    compiler_params=_sc_params,
    scratch_types=(
        pltpu.VMEM((2, CHUNKS, CW), jnp.int32),
        pltpu.VMEM((2, CHUNKS, CW), jnp.float32),
        pltpu.VMEM((CHUNKS, CW), jnp.int32),
        pltpu.VMEM((CHUNKS, CW), jnp.float32),
        pltpu.VMEM((NP,), jnp.float32),
        pltpu.VMEM((NP,), jnp.float32),
        pltpu.VMEM((SLICE_PER_SUB,), jnp.float32),
        pltpu.VMEM((SLICE_PER_SUB,), jnp.float32),
        pltpu.VMEM((SLICE_PER_SUB,), jnp.float32),
        pltpu.VMEM_SHARED((NSUB, NP), jnp.float32),
        pltpu.VMEM_SHARED((NP,), jnp.float32),
    ),
)
def _sc_prep(row_hbm, col_hbm, ew_hbm, wn_hbm,
             row2, ew2, colb, wb, degl, disb, acc, tmpd, disc,
             sh_slots, sh_dis):
    c = lax.axis_index("c")
    s = lax.axis_index("s")
    wid = c * NSUB + s

    pltpu.sync_copy(row_hbm.at[s], row2.at[0])
    pltpu.sync_copy(row_hbm.at[s + NSUB], row2.at[1])
    pltpu.sync_copy(ew_hbm.at[s], ew2.at[0])
    pltpu.sync_copy(ew_hbm.at[s + NSUB], ew2.at[1])
    pltpu.sync_copy(col_hbm.at[wid], colb)

    _zero_flat(degl, NP // 16)

    def deg_body(j, _):
        for t in range(2):
            for k in range(8):
                sl = pl.ds(k * 16, 16)
                plsc.addupdate_scatter(degl, [row2[t, j, sl]], ew2[t, j, sl])
        return 0

    lax.fori_loop(0, CHUNKS, deg_body, 0)

    pltpu.sync_copy(degl, sh_slots.at[s])
    plsc.subcore_barrier()

    base = s * SLICE_PER_SUB
    _zero_flat(acc, SLICE_PER_SUB // 16)

    def red_body(t, _):
        pltpu.sync_copy(sh_slots.at[t, pl.ds(base, SLICE_PER_SUB)], tmpd)

        def add_body(r, _):
            sl = pl.ds(r * 16, 16)
            acc[sl] = acc[sl] + tmpd[sl]
            return 0

        lax.fori_loop(0, SLICE_PER_SUB // 16, add_body, 0)
        return 0

    lax.fori_loop(0, NSUB, red_body, 0)

    def dis_body(r, _):
        sl = pl.ds(r * 16, 16)
        d = acc[sl]
        y = _rsqrt16(jnp.maximum(d, 1e-30))
        disc[sl] = jnp.where(d > 0, y, 0.0)
        return 0

    lax.fori_loop(0, SLICE_PER_SUB // 16, dis_body, 0)
    pltpu.sync_copy(disc, sh_dis.at[pl.ds(base, SLICE_PER_SUB)])
    plsc.subcore_barrier()
    pltpu.sync_copy(sh_dis, disb)

    def wn_body(j, _):
        for k in range(8):
            sl = pl.ds(k * 16, 16)
            dr = plsc.load_gather(disb, [row2[c, j, sl]])
            dc = plsc.load_gather(disb, [colb[j, sl]])
            wb[j, sl] = dr * ew2[c, j, sl] * dc
        return 0

    lax.fori_loop(0, CHUNKS, wn_body, 0)
    pltpu.sync_copy(wb, wn_hbm.at[wid])


@functools.partial(
    pl.kernel,
    out_type=jax.ShapeDtypeStruct((NCORES, NP, HID), jnp.float32),
    mesh=_sc_mesh,
    compiler_params=_sc_params,
    scratch_types=(
        pltpu.VMEM((CHUNKS, CW), jnp.int32),
        pltpu.VMEM((CHUNKS, CW), jnp.int32),
        pltpu.VMEM((CHUNKS, CW), jnp.float32),
        pltpu.VMEM((CW, 16), jnp.float32),
        pltpu.VMEM((NBUF, CW, HID), jnp.float32),
        pltpu.VMEM_SHARED((NP, HID), jnp.float32),
        pltpu.SemaphoreType.DMA((NBUF,)),
    ),
)
def _sc_agg(row_hbm, col_hbm, wn_hbm, v_hbm, agg_hbm,
            rowb, colb, wb, zb, rows_v, sh_agg, sems):
    c = lax.axis_index("c")
    s = lax.axis_index("s")
    wid = c * NSUB + s

    pltpu.sync_copy(row_hbm.at[wid], rowb)
    pltpu.sync_copy(col_hbm.at[wid], colb)
    pltpu.sync_copy(wn_hbm.at[wid], wb)
    _zero_rows(zb, CW)

    _agg_pipeline(s, colb, wb, lambda jj: rowb.at[jj],
                  v_hbm, sh_agg, rows_v, sems, zb)
    _agg_writeout(c, s, sh_agg, agg_hbm)


def _mm1_body(x_ref, w0_ref, w1_ref, y0_ref, y1_ref):
    x = x_ref[...]
    y0_ref[...] = jnp.dot(x, w0_ref[...], preferred_element_type=jnp.float32)
    y1_ref[...] = jnp.dot(x, w1_ref[...], preferred_element_type=jnp.float32)


def _mid_body(y0_ref, agg_ref, b_ref, w0_ref, w1_ref, z0_ref, z1_ref):
    p = agg_ref[0] + agg_ref[1]
    h = jnp.maximum(y0_ref[...] - p + b_ref[0:1, :], 0.0)
    z0_ref[...] = jnp.dot(h, w0_ref[...], preferred_element_type=jnp.float32)
    z1_ref[...] = jnp.dot(h, w1_ref[...], preferred_element_type=jnp.float32)


def _fin_body(z0_ref, agg_ref, b_ref, out_ref):
    o = z0_ref[...] - (agg_ref[0] + agg_ref[1]) + b_ref[0:1, :]
    m = jnp.max(o, axis=1, keepdims=True)
    ex = jnp.exp(o - m)
    out_ref[...] = o - m - jnp.log(jnp.sum(ex, axis=1, keepdims=True))


_RB = 1000


def kernel(x, edge_index, edge_weight, W1, b1, W2, b2):
    row = edge_index[0]
    col = edge_index[1]
    pad = EP - E
    zpad_i = jnp.zeros((pad,), row.dtype)
    rowp = jnp.concatenate([row, zpad_i]).reshape(NTILES, CHUNKS, CW)
    colp = jnp.concatenate([col, zpad_i]).reshape(NTILES, CHUNKS, CW)
    ewp = jnp.concatenate([edge_weight, jnp.zeros((pad,), edge_weight.dtype)])
    ewp = ewp.reshape(NTILES, CHUNKS, CW)
    b1b = jnp.broadcast_to(b1.reshape(1, HID), (8, HID))
    b2b = jnp.broadcast_to(b2.reshape(1, C_OUT), (8, C_OUT))

    grid = N // _RB
    y0, y1 = pl.pallas_call(
        _mm1_body,
        grid=(grid,),
        in_specs=[
            pl.BlockSpec((_RB, F_IN), lambda i: (i, 0)),
            pl.BlockSpec((F_IN, HID), lambda i: (0, 0)),
            pl.BlockSpec((F_IN, HID), lambda i: (0, 0)),
        ],
        out_specs=[
            pl.BlockSpec((_RB, HID), lambda i: (i, 0)),
            pl.BlockSpec((_RB, HID), lambda i: (i, 0)),
        ],
        out_shape=[
            jax.ShapeDtypeStruct((N, HID), jnp.float32),
            jax.ShapeDtypeStruct((N, HID), jnp.float32),
        ],
    )(x, W1[0], W1[1])

    wn = _sc_prep(rowp, colp, ewp)
    agg1 = _sc_agg(rowp, colp, wn, y1)

    z0, z1 = pl.pallas_call(
        _mid_body,
        grid=(grid,),
        in_specs=[
            pl.BlockSpec((_RB, HID), lambda i: (i, 0)),
            pl.BlockSpec((NCORES, _RB, HID), lambda i: (0, i, 0)),
            pl.BlockSpec((8, HID), lambda i: (0, 0)),
            pl.BlockSpec((HID, C_OUT), lambda i: (0, 0)),
            pl.BlockSpec((HID, C_OUT), lambda i: (0, 0)),
        ],
        out_specs=[
            pl.BlockSpec((_RB, C_OUT), lambda i: (i, 0)),
            pl.BlockSpec((_RB, C_OUT), lambda i: (i, 0)),
        ],
        out_shape=[
            jax.ShapeDtypeStruct((N, C_OUT), jnp.float32),
            jax.ShapeDtypeStruct((N, C_OUT), jnp.float32),
        ],
    )(y0, agg1, b1b, W2[0], W2[1])

    agg2 = _sc_agg(rowp, colp, wn, z1)

    out = pl.pallas_call(
        _fin_body,
        grid=(grid,),
        in_specs=[
            pl.BlockSpec((_RB, C_OUT), lambda i: (i, 0)),
            pl.BlockSpec((NCORES, _RB, C_OUT), lambda i: (0, i, 0)),
            pl.BlockSpec((8, C_OUT), lambda i: (0, 0)),
        ],
        out_specs=pl.BlockSpec((_RB, C_OUT), lambda i: (i, 0)),
        out_shape=jax.ShapeDtypeStruct((N, C_OUT), jnp.float32),
    )(z0, agg2, b2b)
    return out
